# baseline (speedup 1.0000x reference)
import jax
import jax.numpy as jnp
from jax.experimental import pallas as pl
from jax.experimental.pallas import tpu as pltpu


def _body(x_ref, cw_ref, cb_ref, g_ref, b2_ref, o_ref):
    u = jnp.maximum(x_ref[...] * cw_ref[...] + cb_ref[...], 0.0)
    acc = jnp.dot(u, g_ref[...], preferred_element_type=jnp.float32)
    o_ref[...] = jnp.maximum(acc + b2_ref[...], 0.0)


def kernel(x, conv_w, conv_b, W0, b0, W1, b1, W2, b2):
    B, I = x.shape
    N = W2.shape[0]
    K = W2.shape[1] // 2
    G = jnp.zeros((I, N), x.dtype).at[33::2, :].set(W2[:, :K].T)
    BM = 512
    out = pl.pallas_call(
        _body,
        grid=(B // BM,),
        in_specs=[
            pl.BlockSpec((BM, I), lambda i: (i, 0)),
            pl.BlockSpec((1, I), lambda i: (0, 0)),
            pl.BlockSpec((1, I), lambda i: (0, 0)),
            pl.BlockSpec((I, N), lambda i: (0, 0)),
            pl.BlockSpec((1, N), lambda i: (0, 0)),
        ],
        out_specs=pl.BlockSpec((BM, N), lambda i: (i, 0)),
        out_shape=jax.ShapeDtypeStruct((B, N), x.dtype),
        compiler_params=pltpu.CompilerParams(
            dimension_semantics=("parallel",),
        ),
    )(x, conv_w[None], conv_b[None], G, b2[None])
    return out


# speedup vs baseline: 1.5327x; 1.5327x over previous
import jax
import jax.numpy as jnp
from jax.experimental import pallas as pl
from jax.experimental.pallas import tpu as pltpu

_BM = 2048


def _body(cw_ref, cb_ref, g_ref, b2_ref, x_hbm, o_hbm, xbuf, obuf, insem, outsem):
    nsteps = x_hbm.shape[0] // _BM

    def in_copy(i, slot):
        return pltpu.make_async_copy(
            x_hbm.at[pl.ds(i * _BM, _BM)], xbuf.at[slot], insem.at[slot])

    def out_copy(i, slot):
        return pltpu.make_async_copy(
            obuf.at[slot], o_hbm.at[pl.ds(i * _BM, _BM)], outsem.at[slot])

    in_copy(0, 0).start()

    def loop(i, carry):
        slot = jax.lax.rem(i, 2)
        @pl.when(i + 1 < nsteps)
        def _():
            in_copy(i + 1, 1 - slot).start()
        in_copy(i, slot).wait()
        u = jnp.maximum(xbuf[slot] * cw_ref[...] + cb_ref[...], 0.0)
        acc = jnp.dot(u, g_ref[...], preferred_element_type=jnp.float32)
        @pl.when(i >= 2)
        def _():
            out_copy(i - 2, slot).wait()
        obuf[slot] = jnp.maximum(acc + b2_ref[...], 0.0)
        out_copy(i, slot).start()
        return carry

    jax.lax.fori_loop(0, nsteps, loop, 0)
    out_copy(nsteps - 2, (nsteps - 2) % 2).wait()
    out_copy(nsteps - 1, (nsteps - 1) % 2).wait()


def kernel(x, conv_w, conv_b, W0, b0, W1, b1, W2, b2):
    B, I = x.shape
    N = W2.shape[0]
    K = W2.shape[1] // 2
    G = jnp.zeros((I, N), x.dtype).at[33::2, :].set(W2[:, :K].T)
    vmem = pl.BlockSpec(memory_space=pltpu.VMEM)
    hbm = pl.BlockSpec(memory_space=pl.ANY)
    out = pl.pallas_call(
        _body,
        in_specs=[vmem, vmem, vmem, vmem, hbm],
        out_specs=hbm,
        out_shape=jax.ShapeDtypeStruct((B, N), x.dtype),
        scratch_shapes=[
            pltpu.VMEM((2, _BM, I), x.dtype),
            pltpu.VMEM((2, _BM, N), x.dtype),
            pltpu.SemaphoreType.DMA((2,)),
            pltpu.SemaphoreType.DMA((2,)),
        ],
    )(conv_w[None], conv_b[None], G, b2[None], x)
    return out


# speedup vs baseline: 1.6298x; 1.0633x over previous
import jax
import jax.numpy as jnp
from jax.experimental import pallas as pl
from jax.experimental.pallas import tpu as pltpu

_BM = 1024
_NSLOT = 4


def _body(cw_ref, cb_ref, g_ref, b2_ref, x_hbm, o_hbm, xbuf, obuf, insem, outsem):
    nsteps = x_hbm.shape[0] // _BM

    def in_copy(i, slot):
        return pltpu.make_async_copy(
            x_hbm.at[pl.ds(i * _BM, _BM)], xbuf.at[slot], insem.at[slot])

    def out_copy(i, slot):
        return pltpu.make_async_copy(
            obuf.at[slot], o_hbm.at[pl.ds(i * _BM, _BM)], outsem.at[slot])

    for j in range(_NSLOT - 1):
        in_copy(j, j).start()

    def loop(i, carry):
        slot = jax.lax.rem(i, _NSLOT)
        @pl.when(i + _NSLOT - 1 < nsteps)
        def _():
            in_copy(i + _NSLOT - 1, jax.lax.rem(i + _NSLOT - 1, _NSLOT)).start()
        in_copy(i, slot).wait()
        u = jnp.maximum(xbuf[slot] * cw_ref[...] + cb_ref[...], 0.0)
        acc = jnp.dot(u, g_ref[...], preferred_element_type=jnp.float32)
        @pl.when(i >= _NSLOT)
        def _():
            out_copy(i - _NSLOT, slot).wait()
        obuf[slot] = jnp.maximum(acc + b2_ref[...], 0.0)
        out_copy(i, slot).start()
        return carry

    jax.lax.fori_loop(0, nsteps, loop, 0)
    for j in range(max(0, nsteps - _NSLOT), nsteps):
        out_copy(j, j % _NSLOT).wait()


def kernel(x, conv_w, conv_b, W0, b0, W1, b1, W2, b2):
    B, I = x.shape
    N = W2.shape[0]
    K = W2.shape[1] // 2
    G = jnp.zeros((I, N), x.dtype).at[33::2, :].set(W2[:, :K].T)
    vmem = pl.BlockSpec(memory_space=pltpu.VMEM)
    hbm = pl.BlockSpec(memory_space=pl.ANY)
    out = pl.pallas_call(
        _body,
        in_specs=[vmem, vmem, vmem, vmem, hbm],
        out_specs=hbm,
        out_shape=jax.ShapeDtypeStruct((B, N), x.dtype),
        scratch_shapes=[
            pltpu.VMEM((_NSLOT, _BM, I), x.dtype),
            pltpu.VMEM((_NSLOT, _BM, N), x.dtype),
            pltpu.SemaphoreType.DMA((_NSLOT,)),
            pltpu.SemaphoreType.DMA((_NSLOT,)),
        ],
    )(conv_w[None], conv_b[None], G, b2[None], x)
    return out
